# baseline (speedup 1.0000x reference)
import functools

import jax
import jax.numpy as jnp
from jax import lax
from jax.experimental import pallas as pl
from jax.experimental.pallas import tpu as pltpu
from jax.experimental.pallas import tpu_sc as plsc

_NUM_CORES = 2
_NUM_SUBCORES = 16
_LANES = 16
_NW = _NUM_CORES * _NUM_SUBCORES


@functools.lru_cache(maxsize=None)
def _build_sc_kernel(B: int, D: int):
    assert B % (_NW * _LANES) == 0
    bpw = B // _NW
    groups = bpw // _LANES

    mesh = plsc.VectorSubcoreMesh(core_axis_name="c", subcore_axis_name="s")

    @functools.partial(
        pl.kernel,
        mesh=mesh,
        out_type=jax.ShapeDtypeStruct((B,), jnp.float32),
        compiler_params=pltpu.CompilerParams(
            needs_layout_passes=False, use_tc_tiling_on_sc=False),
        scratch_types=[
            pltpu.VMEM((bpw,), jnp.int32),
            pltpu.VMEM((bpw,), jnp.int32),
            pltpu.VMEM((bpw, D), jnp.float32),
            pltpu.VMEM((bpw, D), jnp.float32),
            pltpu.VMEM((bpw,), jnp.float32),
            pltpu.VMEM((bpw,), jnp.float32),
            pltpu.VMEM((bpw,), jnp.float32),
            pltpu.SemaphoreType.DMA,
        ],
    )
    def body(uid_hbm, iid_hbm, uemb_hbm, iemb_hbm, ubias_hbm, ibias_hbm,
             out_hbm, uid_v, iid_v, urows, irows, ub_v, ib_v, out_v, sem):
        wid = lax.axis_index("s") * _NUM_CORES + lax.axis_index("c")
        base = wid * bpw

        pltpu.sync_copy(uid_hbm.at[pl.ds(base, bpw)], uid_v)
        pltpu.sync_copy(iid_hbm.at[pl.ds(base, bpw)], iid_v)

        c0 = pltpu.async_copy(uemb_hbm.at[uid_v], urows, sem)
        c1 = pltpu.async_copy(iemb_hbm.at[iid_v], irows, sem)
        c2 = pltpu.async_copy(ubias_hbm.at[uid_v], ub_v, sem)
        c3 = pltpu.async_copy(ibias_hbm.at[iid_v], ib_v, sem)
        c0.wait()
        c1.wait()
        c2.wait()
        c3.wait()

        def group(g, carry):
            gbase = g * _LANES
            rows = gbase + lax.iota(jnp.int32, _LANES)
            acc = ub_v[pl.ds(gbase, _LANES)] + ib_v[pl.ds(gbase, _LANES)]
            for d in range(D):
                col = jnp.full((_LANES,), d, jnp.int32)
                acc = acc + (plsc.load_gather(urows, [rows, col])
                             * plsc.load_gather(irows, [rows, col]))
            out_v[pl.ds(gbase, _LANES)] = acc
            return carry

        lax.fori_loop(0, groups, group, 0)
        pltpu.sync_copy(out_v, out_hbm.at[pl.ds(base, bpw)])

    return body


def kernel(user_id, item_id, user_emb, item_emb, user_bias, item_bias):
    B = user_id.shape[0]
    D = user_emb.shape[1]
    fn = _build_sc_kernel(B, D)
    return fn(
        user_id.astype(jnp.int32),
        item_id.astype(jnp.int32),
        user_emb,
        item_emb,
        user_bias.reshape(-1),
        item_bias.reshape(-1),
    )
